# baseline (speedup 1.0000x reference)
import functools

import jax
import jax.numpy as jnp
from jax.experimental import pallas as pl
from jax.experimental.pallas import tpu as pltpu

_S = 2
_TB = 2048


def _fused_kernel(*refs, l1, l2, s):
    x_refs = refs[:s]
    w_ref, b_ref, y_ref, reg_ref = refs[s:]
    w = w_ref[...]
    b = b_ref[0]
    for j in range(s):
        y_ref[j] = (
            jnp.dot(x_refs[j][0], w, preferred_element_type=jnp.float32) + b
        )
    reg_ref[...] = (l1 * jnp.sum(jnp.abs(w)) + l2 * jnp.sqrt(jnp.sum(w * w))).reshape(
        1, 1
    )


def kernel(x, weight, bias):
    B, D = x.shape
    s, tb = _S, _TB
    b2 = B // s
    grid = (pl.cdiv(b2, tb),)

    x3 = x.reshape(s, b2, D)
    wt = weight.reshape(D, 1)

    def _x_spec(j):
        return pl.BlockSpec((1, tb, D), lambda i, j=j: (j, i, 0))

    y3, reg = pl.pallas_call(
        functools.partial(_fused_kernel, l1=0.01, l2=0.01, s=s),
        grid=grid,
        in_specs=[_x_spec(j) for j in range(s)]
        + [
            pl.BlockSpec((D, 1), lambda i: (0, 0)),
            pl.BlockSpec(memory_space=pltpu.MemorySpace.SMEM),
        ],
        out_specs=[
            pl.BlockSpec((s, tb, 1), lambda i: (0, i, 0)),
            pl.BlockSpec((1, 1), lambda i: (0, 0)),
        ],
        out_shape=[
            jax.ShapeDtypeStruct((s, b2, 1), jnp.float32),
            jax.ShapeDtypeStruct((1, 1), jnp.float32),
        ],
        compiler_params=pltpu.CompilerParams(
            dimension_semantics=("parallel",),
            vmem_limit_bytes=64 * 1024 * 1024,
        ),
    )(*([x3] * s), wt, bias)
    return y3.reshape(B, 1), reg[0, 0]


# speedup vs baseline: 1.0076x; 1.0076x over previous
import functools

import jax
import jax.numpy as jnp
from jax.experimental import pallas as pl
from jax.experimental.pallas import tpu as pltpu

_S = 1
_TB = 8192


def _fused_kernel(*refs, l1, l2, s):
    x_refs = refs[:s]
    w_ref, b_ref, y_ref, reg_ref = refs[s:]
    w = w_ref[...]
    b = b_ref[0]
    for j in range(s):
        y_ref[j] = (
            jnp.dot(x_refs[j][0], w, preferred_element_type=jnp.float32) + b
        )
    reg_ref[...] = (l1 * jnp.sum(jnp.abs(w)) + l2 * jnp.sqrt(jnp.sum(w * w))).reshape(
        1, 1
    )


def kernel(x, weight, bias):
    B, D = x.shape
    s, tb = _S, _TB
    b2 = B // s
    grid = (pl.cdiv(b2, tb),)

    x3 = x.reshape(s, b2, D)
    wt = weight.reshape(D, 1)

    def _x_spec(j):
        return pl.BlockSpec((1, tb, D), lambda i, j=j: (j, i, 0))

    y3, reg = pl.pallas_call(
        functools.partial(_fused_kernel, l1=0.01, l2=0.01, s=s),
        grid=grid,
        in_specs=[_x_spec(j) for j in range(s)]
        + [
            pl.BlockSpec((D, 1), lambda i: (0, 0)),
            pl.BlockSpec(memory_space=pltpu.MemorySpace.SMEM),
        ],
        out_specs=[
            pl.BlockSpec((s, tb, 1), lambda i: (0, i, 0)),
            pl.BlockSpec((1, 1), lambda i: (0, 0)),
        ],
        out_shape=[
            jax.ShapeDtypeStruct((s, b2, 1), jnp.float32),
            jax.ShapeDtypeStruct((1, 1), jnp.float32),
        ],
        compiler_params=pltpu.CompilerParams(
            dimension_semantics=("parallel",),
            vmem_limit_bytes=64 * 1024 * 1024,
        ),
    )(*([x3] * s), wt, bias)
    return y3.reshape(B, 1), reg[0, 0]
